# baseline (speedup 1.0000x reference)
import dataclasses

import jax
import jax.numpy as jnp
from jax import lax
from jax.experimental import pallas as pl
from jax.experimental.pallas import tpu as pltpu
from jax.experimental.pallas import tpu_sc as plsc

_V = 65536
_NNZ = 4294967
_ALPHA = 0.1

_W = 2048
_NWIN = _NNZ // _W
_MAIN = _NWIN * _W
_TA_BASE = _MAIN
_TA_LEN = ((_NNZ - _MAIN) // 16) * 16
_TB_BASE = _TA_BASE + _TA_LEN
_TB_LEN = _NNZ - _TB_BASE

_NC = 2
_NS = 16
_NW = _NC * _NS
_WINS_PER_W = -(-_NWIN // _NW)
_HI_MASK = -65536


def _issue_window(rows_hbm, cols_hbm, vals_hbm, rb, cb, vb, sem, base):
  pltpu.async_copy(rows_hbm.at[pl.ds(base, _W)], rb, sem)
  pltpu.async_copy(cols_hbm.at[pl.ds(base, _W)], cb, sem)
  pltpu.async_copy(vals_hbm.at[pl.ds(base, _W)], vb, sem)


def _wait_window(rows_hbm, cols_hbm, vals_hbm, rb, cb, vb, sem, base):
  pltpu.make_async_copy(rows_hbm.at[pl.ds(base, _W)], rb, sem).wait()
  pltpu.make_async_copy(cols_hbm.at[pl.ds(base, _W)], cb, sem).wait()
  pltpu.make_async_copy(vals_hbm.at[pl.ds(base, _W)], vb, sem).wait()


def _spmv_one(i, rows_ref, cols_ref, vals_ref, packed_ref, acc_ref):
  cols = cols_ref[pl.ds(i, 16)]
  rows = rows_ref[pl.ds(i, 16)]
  vals = vals_ref[pl.ds(i, 16)]
  word = plsc.load_gather(packed_ref, [lax.shift_right_logical(cols, 1)])
  hi = lax.bitwise_and(word, jnp.int32(_HI_MASK))
  lo = lax.shift_left(word, 16)
  g32 = jnp.where(lax.bitwise_and(cols, 1) == 1, hi, lo)
  prod = plsc.bitcast(g32, jnp.float32) * vals
  plsc.addupdate_scatter(acc_ref, [rows], prod)


_UNROLL = 8


def _spmv_vregs(n, rows_ref, cols_ref, vals_ref, packed_ref, acc_ref):
  step = 16 * _UNROLL
  n_main = (n // step) * step
  if n_main:
    @pl.loop(0, n_main, step=step)
    def _(i):
      for u in range(_UNROLL):
        _spmv_one(i + u * 16, rows_ref, cols_ref, vals_ref, packed_ref,
                  acc_ref)
  if n % step:
    @pl.loop(n_main, n, step=16)
    def _(i):
      _spmv_one(i, rows_ref, cols_ref, vals_ref, packed_ref, acc_ref)


def _sc_body(packed_hbm, rows_hbm, cols_hbm, vals_hbm, part_hbm,
             packed_v, acc_v,
             rows0, cols0, vals0, rows1, cols1, vals1,
             rows_s, cols_s, vals_s,
             idx16, r16, c16, v16,
             sem0, sem1):
  cid = lax.axis_index("c")
  sid = lax.axis_index("s")
  wid = sid * _NC + cid

  pltpu.async_copy(packed_hbm, packed_v, sem0)

  @pl.loop(0, _V, step=64)
  def _(i):
    acc_v[pl.ds(i, 16)] = jnp.zeros((16,), jnp.float32)
    acc_v[pl.ds(i + 16, 16)] = jnp.zeros((16,), jnp.float32)
    acc_v[pl.ds(i + 32, 16)] = jnp.zeros((16,), jnp.float32)
    acc_v[pl.ds(i + 48, 16)] = jnp.zeros((16,), jnp.float32)

  pltpu.make_async_copy(packed_hbm, packed_v, sem0).wait()

  bufs = ((rows0, cols0, vals0, sem0), (rows1, cols1, vals1, sem1))

  def win_base(k):
    return jnp.minimum(wid + k * _NW, _NWIN - 1) * _W

  for b in (0, 1):
    rb, cb, vb, sem = bufs[b]
    _issue_window(rows_hbm, cols_hbm, vals_hbm, rb, cb, vb, sem, win_base(b))

  @pl.loop(0, _WINS_PER_W, step=2)
  def _(k):
    for b in (0, 1):
      kk = k + b
      rb, cb, vb, sem = bufs[b]
      _wait_window(rows_hbm, cols_hbm, vals_hbm, rb, cb, vb, sem,
                   win_base(kk))

      @pl.when(wid + kk * _NW < _NWIN)
      def _():
        _spmv_vregs(_W, rb, cb, vb, packed_v, acc_v)

      @pl.when(kk + 2 < _WINS_PER_W)
      def _():
        _issue_window(rows_hbm, cols_hbm, vals_hbm, rb, cb, vb, sem,
                      win_base(kk + 2))

  @pl.when(wid == _NW - 1)
  def _():
    pltpu.sync_copy(rows_hbm.at[pl.ds(_TA_BASE, _TA_LEN)], rows_s)
    pltpu.sync_copy(cols_hbm.at[pl.ds(_TA_BASE, _TA_LEN)], cols_s)
    pltpu.sync_copy(vals_hbm.at[pl.ds(_TA_BASE, _TA_LEN)], vals_s)
    _spmv_vregs(_TA_LEN, rows_s, cols_s, vals_s, packed_v, acc_v)

  @pl.when(wid == _NW - 2)
  def _():
    lane = lax.iota(jnp.int32, 16)
    idx16[...] = jnp.minimum(lane + _TB_BASE, _NNZ - 1)
    pltpu.sync_copy(rows_hbm.at[idx16], r16)
    pltpu.sync_copy(cols_hbm.at[idx16], c16)
    pltpu.sync_copy(vals_hbm.at[idx16], v16)
    v16[...] = jnp.where(lane < _TB_LEN, v16[...], 0.0)
    _spmv_vregs(16, r16, c16, v16, packed_v, acc_v)

  pltpu.sync_copy(acc_v, part_hbm.at[cid, sid])


def _sc_compiler_params():
  cp = pltpu.CompilerParams()
  if "needs_layout_passes" in pltpu.CompilerParams.__dataclass_fields__:
    cp = dataclasses.replace(cp, needs_layout_passes=False)
  return cp


def _sc_spmv(packed, rows, cols, vals):
  kern = pl.kernel(
      _sc_body,
      out_type=jax.ShapeDtypeStruct((_NC, _NS, _V), jnp.float32),
      mesh=plsc.VectorSubcoreMesh(core_axis_name="c", subcore_axis_name="s"),
      compiler_params=_sc_compiler_params(),
      scratch_types=[
          pltpu.VMEM((_V // 2,), jnp.int32),
          pltpu.VMEM((_V,), jnp.float32),
          pltpu.VMEM((_W,), jnp.int32),
          pltpu.VMEM((_W,), jnp.int32),
          pltpu.VMEM((_W,), jnp.float32),
          pltpu.VMEM((_W,), jnp.int32),
          pltpu.VMEM((_W,), jnp.int32),
          pltpu.VMEM((_W,), jnp.float32),
          pltpu.VMEM((_TA_LEN,), jnp.int32),
          pltpu.VMEM((_TA_LEN,), jnp.int32),
          pltpu.VMEM((_TA_LEN,), jnp.float32),
          pltpu.VMEM((16,), jnp.int32),
          pltpu.VMEM((16,), jnp.int32),
          pltpu.VMEM((16,), jnp.int32),
          pltpu.VMEM((16,), jnp.float32),
          pltpu.SemaphoreType.DMA,
          pltpu.SemaphoreType.DMA,
      ],
  )
  return kern(packed, rows, cols, vals)


def _combine_body(p_ref, l_ref, o_ref):
  o_ref[...] = jnp.sum(p_ref[...], axis=0) - _ALPHA * l_ref[...]


@jax.jit
def kernel(logits, S_rows, S_cols, S_vals):
  packed = lax.bitcast_convert_type(
      logits.astype(jnp.bfloat16).reshape(_V // 2, 2), jnp.int32)
  parts = _sc_spmv(packed, S_rows, S_cols, S_vals)
  blk = _V // 16
  out = pl.pallas_call(
      _combine_body,
      grid=(16,),
      in_specs=[
          pl.BlockSpec((_NC * _NS, blk), lambda i: (0, i)),
          pl.BlockSpec((blk,), lambda i: (i,)),
      ],
      out_specs=pl.BlockSpec((blk,), lambda i: (i,)),
      out_shape=jax.ShapeDtypeStruct((_V,), jnp.float32),
  )(parts.reshape(_NC * _NS, _V), logits)
  return out


# speedup vs baseline: 1.6001x; 1.6001x over previous
import dataclasses

import jax
import jax.numpy as jnp
from jax import lax
from jax.experimental import pallas as pl
from jax.experimental.pallas import tpu as pltpu
from jax.experimental.pallas import tpu_sc as plsc

_V = 65536
_NNZ = 4294967
_ALPHA = 0.1

_W = 2048
_NWIN = _NNZ // _W
_MAIN = _NWIN * _W
_TA_BASE = _MAIN
_TA_LEN = ((_NNZ - _MAIN) // 16) * 16
_TB_BASE = _TA_BASE + _TA_LEN
_TB_LEN = _NNZ - _TB_BASE

_NC = 2
_NS = 16
_NW = _NC * _NS
_WINS_PER_W = -(-_NWIN // _NW)
_HI_MASK = -65536


def _issue_window(rows_hbm, cols_hbm, vals_hbm, rb, cb, vb, sem, base):
  pltpu.async_copy(rows_hbm.at[pl.ds(base, _W)], rb, sem)
  pltpu.async_copy(cols_hbm.at[pl.ds(base, _W)], cb, sem)
  pltpu.async_copy(vals_hbm.at[pl.ds(base, _W)], vb, sem)


def _wait_window(rows_hbm, cols_hbm, vals_hbm, rb, cb, vb, sem, base):
  pltpu.make_async_copy(rows_hbm.at[pl.ds(base, _W)], rb, sem).wait()
  pltpu.make_async_copy(cols_hbm.at[pl.ds(base, _W)], cb, sem).wait()
  pltpu.make_async_copy(vals_hbm.at[pl.ds(base, _W)], vb, sem).wait()


def _spmv_one(i, rows_ref, cols_ref, vals_ref, packed_ref, acc_ref):
  cols = cols_ref[pl.ds(i, 16)]
  rows = rows_ref[pl.ds(i, 16)]
  vals = vals_ref[pl.ds(i, 16)]
  word = plsc.load_gather(packed_ref, [lax.shift_right_logical(cols, 1)])
  hi = lax.bitwise_and(word, jnp.int32(_HI_MASK))
  lo = lax.shift_left(word, 16)
  g32 = jnp.where(lax.bitwise_and(cols, 1) == 1, hi, lo)
  prod = plsc.bitcast(g32, jnp.float32) * vals
  plsc.addupdate_scatter(acc_ref, [rows], prod)


_UNROLL = 8


def _spmv_vregs(n, rows_ref, cols_ref, vals_ref, packed_ref, acc_ref):
  @plsc.parallel_loop(0, n, step=16, unroll=_UNROLL)
  def _(i):
    _spmv_one(i, rows_ref, cols_ref, vals_ref, packed_ref, acc_ref)


def _sc_body(packed_hbm, rows_hbm, cols_hbm, vals_hbm, part_hbm,
             packed_v, acc_v,
             rows0, cols0, vals0, rows1, cols1, vals1,
             rows_s, cols_s, vals_s,
             idx16, r16, c16, v16,
             sem0, sem1):
  cid = lax.axis_index("c")
  sid = lax.axis_index("s")
  wid = sid * _NC + cid

  pltpu.async_copy(packed_hbm, packed_v, sem0)

  @pl.loop(0, _V, step=64)
  def _(i):
    acc_v[pl.ds(i, 16)] = jnp.zeros((16,), jnp.float32)
    acc_v[pl.ds(i + 16, 16)] = jnp.zeros((16,), jnp.float32)
    acc_v[pl.ds(i + 32, 16)] = jnp.zeros((16,), jnp.float32)
    acc_v[pl.ds(i + 48, 16)] = jnp.zeros((16,), jnp.float32)

  pltpu.make_async_copy(packed_hbm, packed_v, sem0).wait()

  bufs = ((rows0, cols0, vals0, sem0), (rows1, cols1, vals1, sem1))

  def win_base(k):
    return jnp.minimum(wid + k * _NW, _NWIN - 1) * _W

  for b in (0, 1):
    rb, cb, vb, sem = bufs[b]
    _issue_window(rows_hbm, cols_hbm, vals_hbm, rb, cb, vb, sem, win_base(b))

  @pl.loop(0, _WINS_PER_W, step=2)
  def _(k):
    for b in (0, 1):
      kk = k + b
      rb, cb, vb, sem = bufs[b]
      _wait_window(rows_hbm, cols_hbm, vals_hbm, rb, cb, vb, sem,
                   win_base(kk))

      @pl.when(wid + kk * _NW < _NWIN)
      def _():
        _spmv_vregs(_W, rb, cb, vb, packed_v, acc_v)

      @pl.when(kk + 2 < _WINS_PER_W)
      def _():
        _issue_window(rows_hbm, cols_hbm, vals_hbm, rb, cb, vb, sem,
                      win_base(kk + 2))

  @pl.when(wid == _NW - 1)
  def _():
    pltpu.sync_copy(rows_hbm.at[pl.ds(_TA_BASE, _TA_LEN)], rows_s)
    pltpu.sync_copy(cols_hbm.at[pl.ds(_TA_BASE, _TA_LEN)], cols_s)
    pltpu.sync_copy(vals_hbm.at[pl.ds(_TA_BASE, _TA_LEN)], vals_s)
    _spmv_vregs(_TA_LEN, rows_s, cols_s, vals_s, packed_v, acc_v)

  @pl.when(wid == _NW - 2)
  def _():
    lane = lax.iota(jnp.int32, 16)
    idx16[...] = jnp.minimum(lane + _TB_BASE, _NNZ - 1)
    pltpu.sync_copy(rows_hbm.at[idx16], r16)
    pltpu.sync_copy(cols_hbm.at[idx16], c16)
    pltpu.sync_copy(vals_hbm.at[idx16], v16)
    v16[...] = jnp.where(lane < _TB_LEN, v16[...], 0.0)
    _spmv_vregs(16, r16, c16, v16, packed_v, acc_v)

  pltpu.sync_copy(acc_v, part_hbm.at[cid, sid])


def _sc_compiler_params():
  cp = pltpu.CompilerParams()
  if "needs_layout_passes" in pltpu.CompilerParams.__dataclass_fields__:
    cp = dataclasses.replace(cp, needs_layout_passes=False)
  return cp


def _sc_spmv(packed, rows, cols, vals):
  kern = pl.kernel(
      _sc_body,
      out_type=jax.ShapeDtypeStruct((_NC, _NS, _V), jnp.float32),
      mesh=plsc.VectorSubcoreMesh(core_axis_name="c", subcore_axis_name="s"),
      compiler_params=_sc_compiler_params(),
      scratch_types=[
          pltpu.VMEM((_V // 2,), jnp.int32),
          pltpu.VMEM((_V,), jnp.float32),
          pltpu.VMEM((_W,), jnp.int32),
          pltpu.VMEM((_W,), jnp.int32),
          pltpu.VMEM((_W,), jnp.float32),
          pltpu.VMEM((_W,), jnp.int32),
          pltpu.VMEM((_W,), jnp.int32),
          pltpu.VMEM((_W,), jnp.float32),
          pltpu.VMEM((_TA_LEN,), jnp.int32),
          pltpu.VMEM((_TA_LEN,), jnp.int32),
          pltpu.VMEM((_TA_LEN,), jnp.float32),
          pltpu.VMEM((16,), jnp.int32),
          pltpu.VMEM((16,), jnp.int32),
          pltpu.VMEM((16,), jnp.int32),
          pltpu.VMEM((16,), jnp.float32),
          pltpu.SemaphoreType.DMA,
          pltpu.SemaphoreType.DMA,
      ],
  )
  return kern(packed, rows, cols, vals)


def _combine_body(p_ref, l_ref, o_ref):
  o_ref[...] = jnp.sum(p_ref[...], axis=0) - _ALPHA * l_ref[...]


@jax.jit
def kernel(logits, S_rows, S_cols, S_vals):
  packed = lax.bitcast_convert_type(
      logits.astype(jnp.bfloat16).reshape(_V // 2, 2), jnp.int32)
  parts = _sc_spmv(packed, S_rows, S_cols, S_vals)
  blk = _V // 16
  out = pl.pallas_call(
      _combine_body,
      grid=(16,),
      in_specs=[
          pl.BlockSpec((_NC * _NS, blk), lambda i: (0, i)),
          pl.BlockSpec((blk,), lambda i: (i,)),
      ],
      out_specs=pl.BlockSpec((blk,), lambda i: (i,)),
      out_shape=jax.ShapeDtypeStruct((_V,), jnp.float32),
  )(parts.reshape(_NC * _NS, _V), logits)
  return out


# speedup vs baseline: 1.6380x; 1.0237x over previous
import dataclasses

import jax
import jax.numpy as jnp
from jax import lax
from jax.experimental import pallas as pl
from jax.experimental.pallas import tpu as pltpu
from jax.experimental.pallas import tpu_sc as plsc

_V = 65536
_NNZ = 4294967
_ALPHA = 0.1

_W = 2048
_NWIN = _NNZ // _W
_MAIN = _NWIN * _W
_TA_BASE = _MAIN
_TA_LEN = ((_NNZ - _MAIN) // 16) * 16
_TB_BASE = _TA_BASE + _TA_LEN
_TB_LEN = _NNZ - _TB_BASE

_NC = 2
_NS = 16
_NW = _NC * _NS
_WINS_PER_W = -(-_NWIN // _NW)
_SLICE = _V // _NS
_HI_MASK = -65536


def _issue_window(rows_hbm, cols_hbm, vals_hbm, rb, cb, vb, sem, base):
  pltpu.async_copy(rows_hbm.at[pl.ds(base, _W)], rb, sem)
  pltpu.async_copy(cols_hbm.at[pl.ds(base, _W)], cb, sem)
  pltpu.async_copy(vals_hbm.at[pl.ds(base, _W)], vb, sem)


def _wait_window(rows_hbm, cols_hbm, vals_hbm, rb, cb, vb, sem, base):
  pltpu.make_async_copy(rows_hbm.at[pl.ds(base, _W)], rb, sem).wait()
  pltpu.make_async_copy(cols_hbm.at[pl.ds(base, _W)], cb, sem).wait()
  pltpu.make_async_copy(vals_hbm.at[pl.ds(base, _W)], vb, sem).wait()


def _spmv_one(i, rows_ref, cols_ref, vals_ref, packed_ref, acc_ref):
  cols = cols_ref[pl.ds(i, 16)]
  rows = rows_ref[pl.ds(i, 16)]
  vals = vals_ref[pl.ds(i, 16)]
  word = plsc.load_gather(packed_ref, [lax.shift_right_logical(cols, 1)])
  hi = lax.bitwise_and(word, jnp.int32(_HI_MASK))
  lo = lax.shift_left(word, 16)
  g32 = jnp.where(lax.bitwise_and(cols, 1) == 1, hi, lo)
  prod = plsc.bitcast(g32, jnp.float32) * vals
  plsc.addupdate_scatter(acc_ref, [rows], prod)


_UNROLL = 8


def _spmv_vregs(n, rows_ref, cols_ref, vals_ref, packed_ref, acc_ref):
  @plsc.parallel_loop(0, n, step=16, unroll=_UNROLL)
  def _(i):
    _spmv_one(i, rows_ref, cols_ref, vals_ref, packed_ref, acc_ref)


def _sc_body(packed_hbm, rows_hbm, cols_hbm, vals_hbm, part_hbm, red_hbm,
             packed_v, acc_v, red_v,
             rows0, cols0, vals0, rows1, cols1, vals1,
             rows_s, cols_s, vals_s,
             idx16, r16, c16, v16,
             sem0, sem1):
  cid = lax.axis_index("c")
  sid = lax.axis_index("s")
  wid = sid * _NC + cid

  pltpu.async_copy(packed_hbm, packed_v, sem0)

  @pl.loop(0, _V, step=64)
  def _(i):
    acc_v[pl.ds(i, 16)] = jnp.zeros((16,), jnp.float32)
    acc_v[pl.ds(i + 16, 16)] = jnp.zeros((16,), jnp.float32)
    acc_v[pl.ds(i + 32, 16)] = jnp.zeros((16,), jnp.float32)
    acc_v[pl.ds(i + 48, 16)] = jnp.zeros((16,), jnp.float32)

  pltpu.make_async_copy(packed_hbm, packed_v, sem0).wait()

  bufs = ((rows0, cols0, vals0, sem0), (rows1, cols1, vals1, sem1))

  def win_base(k):
    return jnp.minimum(wid + k * _NW, _NWIN - 1) * _W

  for b in (0, 1):
    rb, cb, vb, sem = bufs[b]
    _issue_window(rows_hbm, cols_hbm, vals_hbm, rb, cb, vb, sem, win_base(b))

  @pl.loop(0, _WINS_PER_W, step=2)
  def _(k):
    for b in (0, 1):
      kk = k + b
      rb, cb, vb, sem = bufs[b]
      _wait_window(rows_hbm, cols_hbm, vals_hbm, rb, cb, vb, sem,
                   win_base(kk))

      @pl.when(wid + kk * _NW < _NWIN)
      def _():
        _spmv_vregs(_W, rb, cb, vb, packed_v, acc_v)

      @pl.when(kk + 2 < _WINS_PER_W)
      def _():
        _issue_window(rows_hbm, cols_hbm, vals_hbm, rb, cb, vb, sem,
                      win_base(kk + 2))

  @pl.when(wid == _NW - 1)
  def _():
    pltpu.sync_copy(rows_hbm.at[pl.ds(_TA_BASE, _TA_LEN)], rows_s)
    pltpu.sync_copy(cols_hbm.at[pl.ds(_TA_BASE, _TA_LEN)], cols_s)
    pltpu.sync_copy(vals_hbm.at[pl.ds(_TA_BASE, _TA_LEN)], vals_s)
    _spmv_vregs(_TA_LEN, rows_s, cols_s, vals_s, packed_v, acc_v)

  @pl.when(wid == _NW - 2)
  def _():
    lane = lax.iota(jnp.int32, 16)
    idx16[...] = jnp.minimum(lane + _TB_BASE, _NNZ - 1)
    pltpu.sync_copy(rows_hbm.at[idx16], r16)
    pltpu.sync_copy(cols_hbm.at[idx16], c16)
    pltpu.sync_copy(vals_hbm.at[idx16], v16)
    v16[...] = jnp.where(lane < _TB_LEN, v16[...], 0.0)
    _spmv_vregs(16, r16, c16, v16, packed_v, acc_v)

  pltpu.sync_copy(acc_v, part_hbm.at[cid, sid])
  plsc.subcore_barrier()

  chunk = sid * _SLICE
  for j in range(_NS):
    pltpu.async_copy(part_hbm.at[cid, j, pl.ds(chunk, _SLICE)],
                     acc_v.at[pl.ds(j * _SLICE, _SLICE)], sem0)
  for j in range(_NS):
    pltpu.make_async_copy(part_hbm.at[cid, j, pl.ds(chunk, _SLICE)],
                          acc_v.at[pl.ds(j * _SLICE, _SLICE)], sem0).wait()

  @plsc.parallel_loop(0, _SLICE, step=16, unroll=4)
  def _(i):
    s = acc_v[pl.ds(i, 16)]
    for j in range(1, _NS):
      s = s + acc_v[pl.ds(j * _SLICE + i, 16)]
    red_v[pl.ds(i, 16)] = s

  pltpu.sync_copy(red_v, red_hbm.at[cid, pl.ds(chunk, _SLICE)])


def _sc_compiler_params():
  cp = pltpu.CompilerParams()
  if "needs_layout_passes" in pltpu.CompilerParams.__dataclass_fields__:
    cp = dataclasses.replace(cp, needs_layout_passes=False)
  return cp


def _sc_spmv(packed, rows, cols, vals):
  kern = pl.kernel(
      _sc_body,
      out_type=(
          jax.ShapeDtypeStruct((_NC, _NS, _V), jnp.float32),
          jax.ShapeDtypeStruct((_NC, _V), jnp.float32),
      ),
      mesh=plsc.VectorSubcoreMesh(core_axis_name="c", subcore_axis_name="s"),
      compiler_params=_sc_compiler_params(),
      scratch_types=[
          pltpu.VMEM((_V // 2,), jnp.int32),
          pltpu.VMEM((_V,), jnp.float32),
          pltpu.VMEM((_SLICE,), jnp.float32),
          pltpu.VMEM((_W,), jnp.int32),
          pltpu.VMEM((_W,), jnp.int32),
          pltpu.VMEM((_W,), jnp.float32),
          pltpu.VMEM((_W,), jnp.int32),
          pltpu.VMEM((_W,), jnp.int32),
          pltpu.VMEM((_W,), jnp.float32),
          pltpu.VMEM((_TA_LEN,), jnp.int32),
          pltpu.VMEM((_TA_LEN,), jnp.int32),
          pltpu.VMEM((_TA_LEN,), jnp.float32),
          pltpu.VMEM((16,), jnp.int32),
          pltpu.VMEM((16,), jnp.int32),
          pltpu.VMEM((16,), jnp.int32),
          pltpu.VMEM((16,), jnp.float32),
          pltpu.SemaphoreType.DMA,
          pltpu.SemaphoreType.DMA,
      ],
  )
  return kern(packed, rows, cols, vals)


def _combine_body(p_ref, l_ref, o_ref):
  o_ref[...] = p_ref[0] + p_ref[1] - _ALPHA * l_ref[...]


@jax.jit
def kernel(logits, S_rows, S_cols, S_vals):
  packed = lax.bitcast_convert_type(
      logits.astype(jnp.bfloat16).reshape(_V // 2, 2), jnp.int32)
  _, red = _sc_spmv(packed, S_rows, S_cols, S_vals)
  out = pl.pallas_call(
      _combine_body,
      out_shape=jax.ShapeDtypeStruct((_V,), jnp.float32),
  )(red, logits)
  return out


# speedup vs baseline: 2.1303x; 1.3006x over previous
import dataclasses

import jax
import jax.numpy as jnp
from jax import lax
from jax.experimental import pallas as pl
from jax.experimental.pallas import tpu as pltpu
from jax.experimental.pallas import tpu_sc as plsc

_V = 65536
_NNZ = 4294967
_ALPHA = 0.1

_W = 2048
_NWIN = _NNZ // _W
_MAIN = _NWIN * _W
_TA_BASE = _MAIN
_TA_LEN = ((_NNZ - _MAIN) // 16) * 16
_TB_BASE = _TA_BASE + _TA_LEN
_TB_LEN = _NNZ - _TB_BASE

_NC = 2
_NS = 16
_NW = _NC * _NS
_WINS_PER_W = -(-_NWIN // _NW)
_SLICE = _V // _NS
_HI_MASK = -65536


def _issue_window(rows_hbm, cols_hbm, vals_hbm, rb, cb, vb, sem, base):
  pltpu.async_copy(rows_hbm.at[pl.ds(base, _W)], rb, sem)
  pltpu.async_copy(cols_hbm.at[pl.ds(base, _W)], cb, sem)
  pltpu.async_copy(vals_hbm.at[pl.ds(base, _W)], vb, sem)


def _wait_window(rows_hbm, cols_hbm, vals_hbm, rb, cb, vb, sem, base):
  pltpu.make_async_copy(rows_hbm.at[pl.ds(base, _W)], rb, sem).wait()
  pltpu.make_async_copy(cols_hbm.at[pl.ds(base, _W)], cb, sem).wait()
  pltpu.make_async_copy(vals_hbm.at[pl.ds(base, _W)], vb, sem).wait()


def _spmv_one(i, rows_ref, cols_ref, vals_ref, packed_ref, acc_ref):
  cols = cols_ref[pl.ds(i, 16)]
  rows = rows_ref[pl.ds(i, 16)]
  vals = vals_ref[pl.ds(i, 16)]
  word = plsc.load_gather(packed_ref,
                          [lax.bitwise_and(cols, jnp.int32(_V // 2 - 1))])
  hi = lax.bitwise_and(word, jnp.int32(_HI_MASK))
  lo = lax.shift_left(word, 16)
  g32 = jnp.where(lax.shift_right_logical(cols, 15) == 1, hi, lo)
  prod = plsc.bitcast(g32, jnp.float32) * vals
  plsc.addupdate_scatter(acc_ref, [rows], prod)


_UNROLL = 8


def _spmv_vregs(n, rows_ref, cols_ref, vals_ref, packed_ref, acc_ref):
  @plsc.parallel_loop(0, n, step=16, unroll=_UNROLL)
  def _(i):
    _spmv_one(i, rows_ref, cols_ref, vals_ref, packed_ref, acc_ref)


def _sc_body(packed_hbm, rows_hbm, cols_hbm, vals_hbm, part_hbm, red_hbm,
             packed_v, acc_v, red_v,
             rows0, cols0, vals0, rows1, cols1, vals1,
             rows_s, cols_s, vals_s,
             idx16, r16, c16, v16,
             sem0, sem1):
  cid = lax.axis_index("c")
  sid = lax.axis_index("s")
  wid = sid * _NC + cid

  pltpu.async_copy(packed_hbm, packed_v, sem0)

  @pl.loop(0, _V, step=64)
  def _(i):
    acc_v[pl.ds(i, 16)] = jnp.zeros((16,), jnp.float32)
    acc_v[pl.ds(i + 16, 16)] = jnp.zeros((16,), jnp.float32)
    acc_v[pl.ds(i + 32, 16)] = jnp.zeros((16,), jnp.float32)
    acc_v[pl.ds(i + 48, 16)] = jnp.zeros((16,), jnp.float32)

  pltpu.make_async_copy(packed_hbm, packed_v, sem0).wait()

  bufs = ((rows0, cols0, vals0, sem0), (rows1, cols1, vals1, sem1))

  def win_base(k):
    return jnp.minimum(wid + k * _NW, _NWIN - 1) * _W

  for b in (0, 1):
    rb, cb, vb, sem = bufs[b]
    _issue_window(rows_hbm, cols_hbm, vals_hbm, rb, cb, vb, sem, win_base(b))

  @pl.loop(0, _WINS_PER_W, step=2)
  def _(k):
    for b in (0, 1):
      kk = k + b
      rb, cb, vb, sem = bufs[b]
      _wait_window(rows_hbm, cols_hbm, vals_hbm, rb, cb, vb, sem,
                   win_base(kk))

      @pl.when(wid + kk * _NW < _NWIN)
      def _():
        _spmv_vregs(_W, rb, cb, vb, packed_v, acc_v)

      @pl.when(kk + 2 < _WINS_PER_W)
      def _():
        _issue_window(rows_hbm, cols_hbm, vals_hbm, rb, cb, vb, sem,
                      win_base(kk + 2))

  @pl.when(wid == _NW - 1)
  def _():
    pltpu.sync_copy(rows_hbm.at[pl.ds(_TA_BASE, _TA_LEN)], rows_s)
    pltpu.sync_copy(cols_hbm.at[pl.ds(_TA_BASE, _TA_LEN)], cols_s)
    pltpu.sync_copy(vals_hbm.at[pl.ds(_TA_BASE, _TA_LEN)], vals_s)
    _spmv_vregs(_TA_LEN, rows_s, cols_s, vals_s, packed_v, acc_v)

  @pl.when(wid == _NW - 2)
  def _():
    lane = lax.iota(jnp.int32, 16)
    idx16[...] = jnp.minimum(lane + _TB_BASE, _NNZ - 1)
    pltpu.sync_copy(rows_hbm.at[idx16], r16)
    pltpu.sync_copy(cols_hbm.at[idx16], c16)
    pltpu.sync_copy(vals_hbm.at[idx16], v16)
    v16[...] = jnp.where(lane < _TB_LEN, v16[...], 0.0)
    _spmv_vregs(16, r16, c16, v16, packed_v, acc_v)

  pltpu.sync_copy(acc_v, part_hbm.at[cid, sid])
  plsc.subcore_barrier()

  chunk = sid * _SLICE
  for j in range(_NS):
    pltpu.async_copy(part_hbm.at[cid, j, pl.ds(chunk, _SLICE)],
                     acc_v.at[pl.ds(j * _SLICE, _SLICE)], sem0)
  for j in range(_NS):
    pltpu.make_async_copy(part_hbm.at[cid, j, pl.ds(chunk, _SLICE)],
                          acc_v.at[pl.ds(j * _SLICE, _SLICE)], sem0).wait()

  @plsc.parallel_loop(0, _SLICE, step=16, unroll=4)
  def _(i):
    s = acc_v[pl.ds(i, 16)]
    for j in range(1, _NS):
      s = s + acc_v[pl.ds(j * _SLICE + i, 16)]
    red_v[pl.ds(i, 16)] = s

  pltpu.sync_copy(red_v, red_hbm.at[cid, pl.ds(chunk, _SLICE)])


def _sc_compiler_params():
  cp = pltpu.CompilerParams()
  if "needs_layout_passes" in pltpu.CompilerParams.__dataclass_fields__:
    cp = dataclasses.replace(cp, needs_layout_passes=False)
  return cp


def _sc_spmv(packed, rows, cols, vals):
  kern = pl.kernel(
      _sc_body,
      out_type=(
          jax.ShapeDtypeStruct((_NC, _NS, _V), jnp.float32),
          jax.ShapeDtypeStruct((_NC, _V), jnp.float32),
      ),
      mesh=plsc.VectorSubcoreMesh(core_axis_name="c", subcore_axis_name="s"),
      compiler_params=_sc_compiler_params(),
      scratch_types=[
          pltpu.VMEM((_V // 2,), jnp.int32),
          pltpu.VMEM((_V,), jnp.float32),
          pltpu.VMEM((_SLICE,), jnp.float32),
          pltpu.VMEM((_W,), jnp.int32),
          pltpu.VMEM((_W,), jnp.int32),
          pltpu.VMEM((_W,), jnp.float32),
          pltpu.VMEM((_W,), jnp.int32),
          pltpu.VMEM((_W,), jnp.int32),
          pltpu.VMEM((_W,), jnp.float32),
          pltpu.VMEM((_TA_LEN,), jnp.int32),
          pltpu.VMEM((_TA_LEN,), jnp.int32),
          pltpu.VMEM((_TA_LEN,), jnp.float32),
          pltpu.VMEM((16,), jnp.int32),
          pltpu.VMEM((16,), jnp.int32),
          pltpu.VMEM((16,), jnp.int32),
          pltpu.VMEM((16,), jnp.float32),
          pltpu.SemaphoreType.DMA,
          pltpu.SemaphoreType.DMA,
      ],
  )
  return kern(packed, rows, cols, vals)


def _combine_body(p_ref, l_ref, o_ref):
  o_ref[...] = p_ref[0] + p_ref[1] - _ALPHA * l_ref[...]


@jax.jit
def kernel(logits, S_rows, S_cols, S_vals):
  bits = lax.add(lax.bitcast_convert_type(logits, jnp.int32),
                 jnp.int32(0x8000))
  packed = lax.bitwise_or(
      lax.shift_right_logical(lax.slice(bits, (0,), (_V // 2,)), 16),
      lax.bitwise_and(lax.slice(bits, (_V // 2,), (_V,)),
                      jnp.int32(_HI_MASK)))
  _, red = _sc_spmv(packed, S_rows, S_cols, S_vals)
  out = pl.pallas_call(
      _combine_body,
      out_shape=jax.ShapeDtypeStruct((_V,), jnp.float32),
  )(red, logits)
  return out


# speedup vs baseline: 2.1316x; 1.0006x over previous
import dataclasses

import jax
import jax.numpy as jnp
from jax import lax
from jax.experimental import pallas as pl
from jax.experimental.pallas import tpu as pltpu
from jax.experimental.pallas import tpu_sc as plsc

_V = 65536
_NNZ = 4294967
_ALPHA = 0.1

_W = 2048
_NWIN = _NNZ // _W
_MAIN = _NWIN * _W
_TA_BASE = _MAIN
_TA_LEN = ((_NNZ - _MAIN) // 16) * 16
_TB_BASE = _TA_BASE + _TA_LEN
_TB_LEN = _NNZ - _TB_BASE

_NC = 2
_NS = 16
_NW = _NC * _NS
_WINS_PER_W = -(-_NWIN // _NW)
_SLICE = _V // _NS
_HI_MASK = -65536


def _issue_window(rows_hbm, cols_hbm, vals_hbm, rb, cb, vb, sem, base):
  pltpu.async_copy(rows_hbm.at[pl.ds(base, _W)], rb, sem)
  pltpu.async_copy(cols_hbm.at[pl.ds(base, _W)], cb, sem)
  pltpu.async_copy(vals_hbm.at[pl.ds(base, _W)], vb, sem)


def _wait_window(rows_hbm, cols_hbm, vals_hbm, rb, cb, vb, sem, base):
  pltpu.make_async_copy(rows_hbm.at[pl.ds(base, _W)], rb, sem).wait()
  pltpu.make_async_copy(cols_hbm.at[pl.ds(base, _W)], cb, sem).wait()
  pltpu.make_async_copy(vals_hbm.at[pl.ds(base, _W)], vb, sem).wait()


def _spmv_one(i, rows_ref, cols_ref, vals_ref, packed_ref, acc_ref):
  cols = cols_ref[pl.ds(i, 16)]
  rows = rows_ref[pl.ds(i, 16)]
  vals = vals_ref[pl.ds(i, 16)]
  word = plsc.load_gather(packed_ref,
                          [lax.bitwise_and(cols, jnp.int32(_V // 2 - 1))])
  hi = lax.bitwise_and(word, jnp.int32(_HI_MASK))
  lo = lax.shift_left(word, 16)
  g32 = jnp.where(lax.shift_right_logical(cols, 15) == 1, hi, lo)
  prod = plsc.bitcast(g32, jnp.float32) * vals
  plsc.addupdate_scatter(acc_ref, [rows], prod)


_UNROLL = 8


def _spmv_vregs(n, rows_ref, cols_ref, vals_ref, packed_ref, acc_ref):
  @plsc.parallel_loop(0, n, step=16, unroll=_UNROLL)
  def _(i):
    _spmv_one(i, rows_ref, cols_ref, vals_ref, packed_ref, acc_ref)


def _sc_body(packed_hbm, rows_hbm, cols_hbm, vals_hbm, part_hbm, red_hbm,
             packed_v, acc_v, red_v,
             rows0, cols0, vals0, rows1, cols1, vals1,
             idx16, r16, c16, v16,
             sem0, sem1):
  cid = lax.axis_index("c")
  sid = lax.axis_index("s")
  wid = sid * _NC + cid

  pltpu.async_copy(packed_hbm, packed_v, sem0)

  @pl.loop(0, _V, step=64)
  def _(i):
    acc_v[pl.ds(i, 16)] = jnp.zeros((16,), jnp.float32)
    acc_v[pl.ds(i + 16, 16)] = jnp.zeros((16,), jnp.float32)
    acc_v[pl.ds(i + 32, 16)] = jnp.zeros((16,), jnp.float32)
    acc_v[pl.ds(i + 48, 16)] = jnp.zeros((16,), jnp.float32)

  pltpu.make_async_copy(packed_hbm, packed_v, sem0).wait()

  bufs = ((rows0, cols0, vals0, sem0), (rows1, cols1, vals1, sem1))

  def win_base(k):
    return jnp.minimum(wid + k * _NW, _NWIN - 1) * _W

  for b in (0, 1):
    rb, cb, vb, sem = bufs[b]
    _issue_window(rows_hbm, cols_hbm, vals_hbm, rb, cb, vb, sem, win_base(b))

  @pl.loop(0, _WINS_PER_W, step=2)
  def _(k):
    for b in (0, 1):
      kk = k + b
      rb, cb, vb, sem = bufs[b]
      _wait_window(rows_hbm, cols_hbm, vals_hbm, rb, cb, vb, sem,
                   win_base(kk))

      @pl.when(wid + kk * _NW < _NWIN)
      def _():
        _spmv_vregs(_W, rb, cb, vb, packed_v, acc_v)

      @pl.when(kk + 2 < _WINS_PER_W)
      def _():
        _issue_window(rows_hbm, cols_hbm, vals_hbm, rb, cb, vb, sem,
                      win_base(kk + 2))

  @pl.when(wid == _NW - 1)
  def _():
    pltpu.sync_copy(rows_hbm.at[pl.ds(_TA_BASE, _TA_LEN)],
                    rows0.at[pl.ds(0, _TA_LEN)])
    pltpu.sync_copy(cols_hbm.at[pl.ds(_TA_BASE, _TA_LEN)],
                    cols0.at[pl.ds(0, _TA_LEN)])
    pltpu.sync_copy(vals_hbm.at[pl.ds(_TA_BASE, _TA_LEN)],
                    vals0.at[pl.ds(0, _TA_LEN)])
    _spmv_vregs(_TA_LEN, rows0, cols0, vals0, packed_v, acc_v)

  @pl.when(wid == _NW - 2)
  def _():
    lane = lax.iota(jnp.int32, 16)
    idx16[...] = jnp.minimum(lane + _TB_BASE, _NNZ - 1)
    pltpu.sync_copy(rows_hbm.at[idx16], r16)
    pltpu.sync_copy(cols_hbm.at[idx16], c16)
    pltpu.sync_copy(vals_hbm.at[idx16], v16)
    v16[...] = jnp.where(lane < _TB_LEN, v16[...], 0.0)
    _spmv_vregs(16, r16, c16, v16, packed_v, acc_v)

  pltpu.sync_copy(acc_v, part_hbm.at[cid, sid])
  plsc.subcore_barrier()

  chunk = sid * _SLICE
  for j in range(_NS):
    pltpu.async_copy(part_hbm.at[cid, j, pl.ds(chunk, _SLICE)],
                     acc_v.at[pl.ds(j * _SLICE, _SLICE)], sem0)
  for j in range(_NS):
    pltpu.make_async_copy(part_hbm.at[cid, j, pl.ds(chunk, _SLICE)],
                          acc_v.at[pl.ds(j * _SLICE, _SLICE)], sem0).wait()

  @plsc.parallel_loop(0, _SLICE, step=16, unroll=4)
  def _(i):
    s = acc_v[pl.ds(i, 16)]
    for j in range(1, _NS):
      s = s + acc_v[pl.ds(j * _SLICE + i, 16)]
    red_v[pl.ds(i, 16)] = s

  pltpu.sync_copy(red_v, red_hbm.at[cid, pl.ds(chunk, _SLICE)])


def _sc_compiler_params():
  cp = pltpu.CompilerParams()
  if "needs_layout_passes" in pltpu.CompilerParams.__dataclass_fields__:
    cp = dataclasses.replace(cp, needs_layout_passes=False)
  return cp


def _sc_spmv(packed, rows, cols, vals):
  kern = pl.kernel(
      _sc_body,
      out_type=(
          jax.ShapeDtypeStruct((_NC, _NS, _V), jnp.float32),
          jax.ShapeDtypeStruct((_NC, _V), jnp.float32),
      ),
      mesh=plsc.VectorSubcoreMesh(core_axis_name="c", subcore_axis_name="s"),
      compiler_params=_sc_compiler_params(),
      scratch_types=[
          pltpu.VMEM((_V // 2,), jnp.int32),
          pltpu.VMEM((_V,), jnp.float32),
          pltpu.VMEM((_SLICE,), jnp.float32),
          pltpu.VMEM((_W,), jnp.int32),
          pltpu.VMEM((_W,), jnp.int32),
          pltpu.VMEM((_W,), jnp.float32),
          pltpu.VMEM((_W,), jnp.int32),
          pltpu.VMEM((_W,), jnp.int32),
          pltpu.VMEM((_W,), jnp.float32),
          pltpu.VMEM((16,), jnp.int32),
          pltpu.VMEM((16,), jnp.int32),
          pltpu.VMEM((16,), jnp.int32),
          pltpu.VMEM((16,), jnp.float32),
          pltpu.SemaphoreType.DMA,
          pltpu.SemaphoreType.DMA,
      ],
  )
  return kern(packed, rows, cols, vals)


def _combine_body(p_ref, l_ref, o_ref):
  o_ref[...] = p_ref[0] + p_ref[1] - _ALPHA * l_ref[...]


@jax.jit
def kernel(logits, S_rows, S_cols, S_vals):
  bits = lax.add(lax.bitcast_convert_type(logits, jnp.int32),
                 jnp.int32(0x8000))
  packed = lax.bitwise_or(
      lax.shift_right_logical(lax.slice(bits, (0,), (_V // 2,)), 16),
      lax.bitwise_and(lax.slice(bits, (_V // 2,), (_V,)),
                      jnp.int32(_HI_MASK)))
  _, red = _sc_spmv(packed, S_rows, S_cols, S_vals)
  out = pl.pallas_call(
      _combine_body,
      out_shape=jax.ShapeDtypeStruct((_V,), jnp.float32),
  )(red, logits)
  return out


# speedup vs baseline: 2.4330x; 1.1414x over previous
import dataclasses

import jax
import jax.numpy as jnp
from jax import lax
from jax.experimental import pallas as pl
from jax.experimental.pallas import tpu as pltpu
from jax.experimental.pallas import tpu_sc as plsc

_V = 65536
_NNZ = 4294967
_ALPHA = 0.1

_W = 2048
_NWIN = _NNZ // _W
_MAIN = _NWIN * _W
_TA_BASE = _MAIN
_TA_LEN = ((_NNZ - _MAIN) // 16) * 16
_TB_BASE = _TA_BASE + _TA_LEN
_TB_LEN = _NNZ - _TB_BASE

_NC = 2
_NS = 16
_NW = _NC * _NS
_WINS_PER_W = -(-_NWIN // _NW)
_SLICE = _V // _NS
_HI_MASK = -65536


def _issue_window(rows_hbm, cols_hbm, vals_hbm, rb, cb, vb, sem, base):
  pltpu.async_copy(rows_hbm.at[pl.ds(base, _W)], rb, sem)
  pltpu.async_copy(cols_hbm.at[pl.ds(base, _W)], cb, sem)
  pltpu.async_copy(vals_hbm.at[pl.ds(base, _W)], vb, sem)


def _wait_window(rows_hbm, cols_hbm, vals_hbm, rb, cb, vb, sem, base):
  pltpu.make_async_copy(rows_hbm.at[pl.ds(base, _W)], rb, sem).wait()
  pltpu.make_async_copy(cols_hbm.at[pl.ds(base, _W)], cb, sem).wait()
  pltpu.make_async_copy(vals_hbm.at[pl.ds(base, _W)], vb, sem).wait()


def _spmv_one(i, rows_ref, cols_ref, vals_ref, packed_ref, acc_ref):
  cols = cols_ref[pl.ds(i, 16)]
  rows = rows_ref[pl.ds(i, 16)]
  vals = vals_ref[pl.ds(i, 16)]
  word = plsc.load_gather(packed_ref,
                          [lax.bitwise_and(cols, jnp.int32(_V // 2 - 1))])
  hi = lax.bitwise_and(word, jnp.int32(_HI_MASK))
  lo = lax.shift_left(word, 16)
  g32 = jnp.where(lax.shift_right_logical(cols, 15) == 1, hi, lo)
  prod = plsc.bitcast(g32, jnp.float32) * vals
  plsc.addupdate_scatter(acc_ref, [rows], prod)


_UNROLL = 8


def _spmv_vregs(n, rows_ref, cols_ref, vals_ref, packed_ref, acc_ref):
  @plsc.parallel_loop(0, n, step=16, unroll=_UNROLL)
  def _(i):
    _spmv_one(i, rows_ref, cols_ref, vals_ref, packed_ref, acc_ref)


def _sc_body(packed_hbm, rows_hbm, cols_hbm, vals_hbm, part_hbm, red_hbm,
             packed_v, acc_v, red_v,
             rows0, cols0, vals0, rows1, cols1, vals1, rows2, cols2, vals2,
             idx16, r16, c16, v16,
             sem0, sem1, sem2):
  cid = lax.axis_index("c")
  sid = lax.axis_index("s")
  wid = sid * _NC + cid

  pltpu.async_copy(packed_hbm, packed_v, sem0)

  @pl.loop(0, _V, step=64)
  def _(i):
    acc_v[pl.ds(i, 16)] = jnp.zeros((16,), jnp.float32)
    acc_v[pl.ds(i + 16, 16)] = jnp.zeros((16,), jnp.float32)
    acc_v[pl.ds(i + 32, 16)] = jnp.zeros((16,), jnp.float32)
    acc_v[pl.ds(i + 48, 16)] = jnp.zeros((16,), jnp.float32)

  pltpu.make_async_copy(packed_hbm, packed_v, sem0).wait()

  bufs = ((rows0, cols0, vals0, sem0), (rows1, cols1, vals1, sem1),
          (rows2, cols2, vals2, sem2))
  nbuf = len(bufs)

  def win_base(k):
    return jnp.minimum(wid + k * _NW, _NWIN - 1) * _W

  for b in range(nbuf):
    rb, cb, vb, sem = bufs[b]
    _issue_window(rows_hbm, cols_hbm, vals_hbm, rb, cb, vb, sem, win_base(b))

  @pl.loop(0, _WINS_PER_W, step=nbuf)
  def _(k):
    for b in range(nbuf):
      kk = k + b
      rb, cb, vb, sem = bufs[b]
      _wait_window(rows_hbm, cols_hbm, vals_hbm, rb, cb, vb, sem,
                   win_base(kk))

      @pl.when(wid + kk * _NW < _NWIN)
      def _():
        _spmv_vregs(_W, rb, cb, vb, packed_v, acc_v)

      @pl.when(kk + nbuf < _WINS_PER_W)
      def _():
        _issue_window(rows_hbm, cols_hbm, vals_hbm, rb, cb, vb, sem,
                      win_base(kk + nbuf))

  @pl.when(wid == _NW - 1)
  def _():
    pltpu.sync_copy(rows_hbm.at[pl.ds(_TA_BASE, _TA_LEN)],
                    rows0.at[pl.ds(0, _TA_LEN)])
    pltpu.sync_copy(cols_hbm.at[pl.ds(_TA_BASE, _TA_LEN)],
                    cols0.at[pl.ds(0, _TA_LEN)])
    pltpu.sync_copy(vals_hbm.at[pl.ds(_TA_BASE, _TA_LEN)],
                    vals0.at[pl.ds(0, _TA_LEN)])
    _spmv_vregs(_TA_LEN, rows0, cols0, vals0, packed_v, acc_v)

  @pl.when(wid == _NW - 2)
  def _():
    lane = lax.iota(jnp.int32, 16)
    idx16[...] = jnp.minimum(lane + _TB_BASE, _NNZ - 1)
    pltpu.sync_copy(rows_hbm.at[idx16], r16)
    pltpu.sync_copy(cols_hbm.at[idx16], c16)
    pltpu.sync_copy(vals_hbm.at[idx16], v16)
    v16[...] = jnp.where(lane < _TB_LEN, v16[...], 0.0)
    _spmv_vregs(16, r16, c16, v16, packed_v, acc_v)

  pltpu.sync_copy(acc_v, part_hbm.at[cid, sid])
  plsc.subcore_barrier()

  chunk = sid * _SLICE
  for j in range(_NS):
    pltpu.async_copy(part_hbm.at[cid, j, pl.ds(chunk, _SLICE)],
                     acc_v.at[pl.ds(j * _SLICE, _SLICE)], sem0)
  for j in range(_NS):
    pltpu.make_async_copy(part_hbm.at[cid, j, pl.ds(chunk, _SLICE)],
                          acc_v.at[pl.ds(j * _SLICE, _SLICE)], sem0).wait()

  @plsc.parallel_loop(0, _SLICE, step=16, unroll=4)
  def _(i):
    s = acc_v[pl.ds(i, 16)]
    for j in range(1, _NS):
      s = s + acc_v[pl.ds(j * _SLICE + i, 16)]
    red_v[pl.ds(i, 16)] = s

  pltpu.sync_copy(red_v, red_hbm.at[cid, pl.ds(chunk, _SLICE)])


def _sc_compiler_params():
  cp = pltpu.CompilerParams()
  if "needs_layout_passes" in pltpu.CompilerParams.__dataclass_fields__:
    cp = dataclasses.replace(cp, needs_layout_passes=False)
  return cp


def _sc_spmv(packed, rows, cols, vals):
  kern = pl.kernel(
      _sc_body,
      out_type=(
          jax.ShapeDtypeStruct((_NC, _NS, _V), jnp.float32),
          jax.ShapeDtypeStruct((_NC, _V), jnp.float32),
      ),
      mesh=plsc.VectorSubcoreMesh(core_axis_name="c", subcore_axis_name="s"),
      compiler_params=_sc_compiler_params(),
      scratch_types=[
          pltpu.VMEM((_V // 2,), jnp.int32),
          pltpu.VMEM((_V,), jnp.float32),
          pltpu.VMEM((_SLICE,), jnp.float32),
          pltpu.VMEM((_W,), jnp.int32),
          pltpu.VMEM((_W,), jnp.int32),
          pltpu.VMEM((_W,), jnp.float32),
          pltpu.VMEM((_W,), jnp.int32),
          pltpu.VMEM((_W,), jnp.int32),
          pltpu.VMEM((_W,), jnp.float32),
          pltpu.VMEM((_W,), jnp.int32),
          pltpu.VMEM((_W,), jnp.int32),
          pltpu.VMEM((_W,), jnp.float32),
          pltpu.VMEM((16,), jnp.int32),
          pltpu.VMEM((16,), jnp.int32),
          pltpu.VMEM((16,), jnp.int32),
          pltpu.VMEM((16,), jnp.float32),
          pltpu.SemaphoreType.DMA,
          pltpu.SemaphoreType.DMA,
          pltpu.SemaphoreType.DMA,
      ],
  )
  return kern(packed, rows, cols, vals)


def _combine_body(p_ref, l_ref, o_ref):
  o_ref[...] = p_ref[0] + p_ref[1] - _ALPHA * l_ref[...]


@jax.jit
def kernel(logits, S_rows, S_cols, S_vals):
  bits = lax.add(lax.bitcast_convert_type(logits, jnp.int32),
                 jnp.int32(0x8000))
  packed = lax.bitwise_or(
      lax.shift_right_logical(lax.slice(bits, (0,), (_V // 2,)), 16),
      lax.bitwise_and(lax.slice(bits, (_V // 2,), (_V,)),
                      jnp.int32(_HI_MASK)))
  _, red = _sc_spmv(packed, S_rows, S_cols, S_vals)
  out = pl.pallas_call(
      _combine_body,
      out_shape=jax.ShapeDtypeStruct((_V,), jnp.float32),
  )(red, logits)
  return out
